# Initial kernel scaffold; baseline (speedup 1.0000x reference)
import jax
import jax.numpy as jnp
from jax.experimental import pallas as pl


def kernel(inputs, table):
    raise NotImplementedError("write your pallas kernel here")


# speedup vs baseline: 7.8431x; 7.8431x over previous
import jax
import jax.numpy as jnp
from jax.experimental import pallas as pl
from jax.experimental.pallas import tpu as pltpu

_B, _S, _D = 16384, 200, 64
_ROWS = 64


def _body(idx_ref, tab_ref, out_ref):
    idx = idx_ref[...]
    oh = (idx[:, :, None] == jax.lax.broadcasted_iota(jnp.int32, (1, 1, 8), 2)
          ).astype(jnp.float32)
    out_ref[...] = jax.lax.dot_general(
        oh, tab_ref[...], (((2,), (0,)), ((), ())),
        preferred_element_type=jnp.float32)


def kernel(inputs, table):
    idx = inputs.astype(jnp.int32)
    tab = jnp.zeros((8, _D), jnp.float32).at[:7].set(table)
    grid = (_B // _ROWS,)
    out = pl.pallas_call(
        _body,
        grid=grid,
        in_specs=[
            pl.BlockSpec((_ROWS, _S), lambda i: (i, 0)),
            pl.BlockSpec((8, _D), lambda i: (0, 0)),
        ],
        out_specs=pl.BlockSpec((_ROWS, _S, _D), lambda i: (i, 0, 0)),
        out_shape=jax.ShapeDtypeStruct((_B, _S, _D), jnp.float32),
    )(idx, tab)
    return out
